# baseline (speedup 1.0000x reference)
import functools

import jax
import jax.numpy as jnp
from jax import lax
from jax.experimental import pallas as pl
from jax.experimental.pallas import tpu as pltpu
from jax.experimental.pallas import tpu_sc as plsc

NUM_EXPERTS = 16
TOP_K = 2
HIDDEN_DIM = 2048
NUM_TOKENS = 16384

BLOCK_N = 1024
NBUF = 4

_SC_INFO = plsc.get_sparse_core_info()
_NC = _SC_INFO.num_cores
_NS = _SC_INFO.num_subcores
_NW = _NC * _NS
_CHUNK = NUM_TOKENS // _NW
_GROUPS = _CHUNK // 16


def _probs_block(x_hbm, c_ref, probs_ref, xbuf, sems):
    i = pl.program_id(0)
    nchunk = pl.num_programs(0)

    def start(chunk):
        slot = jax.lax.rem(chunk, NBUF)
        pltpu.make_async_copy(
            x_hbm.at[pl.ds(chunk * BLOCK_N, BLOCK_N), :],
            xbuf.at[slot],
            sems.at[slot],
        ).start()

    @pl.when(i == 0)
    def _prologue():
        for j in range(NBUF - 1):
            start(j)

    @pl.when(i + NBUF - 1 < nchunk)
    def _steady():
        start(i + NBUF - 1)

    slot = jax.lax.rem(i, NBUF)
    pltpu.make_async_copy(
        x_hbm.at[pl.ds(i * BLOCK_N, BLOCK_N), :],
        xbuf.at[slot],
        sems.at[slot],
    ).wait()

    x = xbuf[slot]
    c = c_ref[...]

    x2 = jnp.sum(x * x, axis=1, keepdims=True)
    c2 = jnp.sum(c * c, axis=1)[None, :]
    xc = jax.lax.dot_general(
        x, c, (((1,), (1,)), ((), ())),
        preferred_element_type=jnp.float32)
    d2 = jnp.maximum(x2 + c2 - 2.0 * xc, 0.0)
    neg_d = -jnp.sqrt(d2)

    m = jnp.max(neg_d, axis=1, keepdims=True)
    e = jnp.exp(neg_d - m)
    s = jnp.sum(e, axis=1, keepdims=True)
    probs_ref[...] = e / s


def _probs_call(x, centroids):
    n, d = x.shape
    e = centroids.shape[0]
    return pl.pallas_call(
        _probs_block,
        grid=(n // BLOCK_N,),
        in_specs=[
            pl.BlockSpec(memory_space=pltpu.MemorySpace.HBM),
            pl.BlockSpec((e, d), lambda i: (0, 0)),
        ],
        out_specs=pl.BlockSpec((BLOCK_N, e), lambda i: (i, 0)),
        out_shape=jax.ShapeDtypeStruct((n, e), jnp.float32),
        scratch_shapes=[
            pltpu.VMEM((NBUF, BLOCK_N, d), jnp.float32),
            pltpu.SemaphoreType.DMA((NBUF,)),
        ],
        compiler_params=pltpu.CompilerParams(
            dimension_semantics=("arbitrary",),
        ),
    )(x, centroids)


def _topk_body(probs_hbm, idx_hbm, tkp_hbm, p_v, idx_v, tkp_v):
    wid = lax.axis_index("s") * _NC + lax.axis_index("c")
    base = wid * _CHUNK
    pltpu.sync_copy(probs_hbm.at[pl.ds(base, _CHUNK), :], p_v)

    lanes = lax.iota(jnp.int32, 16)

    def group(g, carry):
        rows = g * 16 + lanes
        m1 = jnp.full((16,), -1.0, jnp.float32)
        m2 = jnp.full((16,), -1.0, jnp.float32)
        i1 = jnp.zeros((16,), jnp.int32)
        i2 = jnp.zeros((16,), jnp.int32)
        for e in range(NUM_EXPERTS):
            v = plsc.load_gather(p_v, [rows, jnp.full((16,), e, jnp.int32)])
            is1 = v > m1
            is2 = v > m2
            m2 = jnp.where(is1, m1, jnp.where(is2, v, m2))
            i2 = jnp.where(is1, i1, jnp.where(is2, e, i2))
            m1 = jnp.where(is1, v, m1)
            i1 = jnp.where(is1, e, i1)
        s = m1 + m2
        zero = jnp.zeros((16,), jnp.int32)
        one = jnp.ones((16,), jnp.int32)
        plsc.store_scatter(idx_v, [rows, zero], i1)
        plsc.store_scatter(idx_v, [rows, one], i2)
        plsc.store_scatter(tkp_v, [rows, zero], m1 / s)
        plsc.store_scatter(tkp_v, [rows, one], m2 / s)
        return carry

    lax.fori_loop(0, _GROUPS, group, 0)

    pltpu.sync_copy(idx_v, idx_hbm.at[pl.ds(base, _CHUNK), :])
    pltpu.sync_copy(tkp_v, tkp_hbm.at[pl.ds(base, _CHUNK), :])


def _topk_call(probs):
    n = probs.shape[0]
    mesh = plsc.VectorSubcoreMesh(core_axis_name="c", subcore_axis_name="s")
    f = pl.kernel(
        _topk_body,
        mesh=mesh,
        out_type=(
            jax.ShapeDtypeStruct((n, TOP_K), jnp.int32),
            jax.ShapeDtypeStruct((n, TOP_K), jnp.float32),
        ),
        scratch_types=[
            pltpu.VMEM((_CHUNK, NUM_EXPERTS), jnp.float32),
            pltpu.VMEM((_CHUNK, TOP_K), jnp.int32),
            pltpu.VMEM((_CHUNK, TOP_K), jnp.float32),
        ],
        compiler_params=pltpu.CompilerParams(needs_layout_passes=False, use_tc_tiling_on_sc=False),
    )
    return f(probs)


@jax.jit
def kernel(x, centroids):
    probs = _probs_call(x, centroids)
    idx, tkp = _topk_call(probs)
    return (idx, tkp, probs)


# speedup vs baseline: 1.4533x; 1.4533x over previous
import functools

import jax
import jax.numpy as jnp
from jax.experimental import pallas as pl
from jax.experimental.pallas import tpu as pltpu

NUM_EXPERTS = 16
TOP_K = 2
HIDDEN_DIM = 2048
NUM_TOKENS = 16384

BLOCK_N = 1024
NBUF = 4


def _router_block(x_hbm, c_ref, idx_ref, tkp_ref, probs_ref, xbuf, sems, sems2):
    i = pl.program_id(0)
    nchunk = pl.num_programs(0)

    half = BLOCK_N // 2

    def start(chunk):
        slot = jax.lax.rem(chunk, NBUF)
        pltpu.make_async_copy(
            x_hbm.at[pl.ds(chunk * BLOCK_N, half), :],
            xbuf.at[slot, pl.ds(0, half)],
            sems.at[slot],
        ).start()
        pltpu.make_async_copy(
            x_hbm.at[pl.ds(chunk * BLOCK_N + half, half), :],
            xbuf.at[slot, pl.ds(half, half)],
            sems2.at[slot],
        ).start()

    @pl.when(i == 0)
    def _prologue():
        for j in range(NBUF - 1):
            start(j)

    @pl.when(i + NBUF - 1 < nchunk)
    def _steady():
        start(i + NBUF - 1)

    slot = jax.lax.rem(i, NBUF)
    pltpu.make_async_copy(
        x_hbm.at[pl.ds(i * BLOCK_N, half), :],
        xbuf.at[slot, pl.ds(0, half)],
        sems.at[slot],
    ).wait()
    pltpu.make_async_copy(
        x_hbm.at[pl.ds(i * BLOCK_N + half, half), :],
        xbuf.at[slot, pl.ds(half, half)],
        sems2.at[slot],
    ).wait()

    x = xbuf[slot]
    c = c_ref[...]
    b = x.shape[0]

    x2 = jnp.sum(x * x, axis=1, keepdims=True)
    c2 = jnp.sum(c * c, axis=1)[None, :]
    xc = jax.lax.dot_general(
        x, c, (((1,), (1,)), ((), ())),
        preferred_element_type=jnp.float32)
    d2 = jnp.maximum(x2 + c2 - 2.0 * xc, 0.0)
    neg_d = -jnp.sqrt(d2)

    m = jnp.max(neg_d, axis=1, keepdims=True)
    e = jnp.exp(neg_d - m)
    s = jnp.sum(e, axis=1, keepdims=True)
    probs = e / s
    probs_ref[...] = probs

    lane = jax.lax.broadcasted_iota(jnp.int32, (b, NUM_EXPERTS), 1)
    m1 = jnp.max(probs, axis=1, keepdims=True)
    i1 = jnp.min(jnp.where(probs == m1, lane, NUM_EXPERTS),
                 axis=1, keepdims=True)
    masked = jnp.where(lane == i1, -jnp.float32(1.0), probs)
    m2 = jnp.max(masked, axis=1, keepdims=True)
    i2 = jnp.min(jnp.where(masked == m2, lane, NUM_EXPERTS),
                 axis=1, keepdims=True)

    denom = m1 + m2
    idx_ref[...] = jnp.concatenate([i1, i2], axis=1)
    tkp_ref[...] = jnp.concatenate([m1 / denom, m2 / denom], axis=1)


@jax.jit
def kernel(x, centroids):
    n, d = x.shape
    e = centroids.shape[0]
    grid = (n // BLOCK_N,)
    out_shapes = (
        jax.ShapeDtypeStruct((n, TOP_K), jnp.int32),
        jax.ShapeDtypeStruct((n, TOP_K), jnp.float32),
        jax.ShapeDtypeStruct((n, e), jnp.float32),
    )
    return pl.pallas_call(
        _router_block,
        grid=grid,
        in_specs=[
            pl.BlockSpec(memory_space=pltpu.MemorySpace.HBM),
            pl.BlockSpec((e, d), lambda i: (0, 0)),
        ],
        out_specs=(
            pl.BlockSpec((BLOCK_N, TOP_K), lambda i: (i, 0)),
            pl.BlockSpec((BLOCK_N, TOP_K), lambda i: (i, 0)),
            pl.BlockSpec((BLOCK_N, e), lambda i: (i, 0)),
        ),
        out_shape=out_shapes,
        scratch_shapes=[
            pltpu.VMEM((NBUF, BLOCK_N, d), jnp.float32),
            pltpu.SemaphoreType.DMA((NBUF,)),
            pltpu.SemaphoreType.DMA((NBUF,)),
        ],
        compiler_params=pltpu.CompilerParams(
            dimension_semantics=("arbitrary",),
        ),
    )(x, centroids)
